# baseline (speedup 1.0000x reference)
import functools

import jax
import jax.numpy as jnp
from jax import lax
from jax.experimental import pallas as pl
from jax.experimental.pallas import tpu as pltpu
from jax.experimental.pallas import tpu_sc as plsc

D_MODEL = 128
NC = 2
NS = 16
NW = NC * NS
CHUNK = 64
NBUF = 14
GDEPTH = 10


@functools.partial(jax.jit, static_argnames=("batch",))
def _emb_lookup(table, idx_flat, *, batch):
    b_per_w = batch // NW
    n_chunks = b_per_w // CHUNK
    assert n_chunks * CHUNK == b_per_w and n_chunks >= 2 * NBUF
    mesh = plsc.VectorSubcoreMesh(
        core_axis_name="c", subcore_axis_name="s",
        num_cores=NC, num_subcores=NS)

    @functools.partial(
        pl.kernel,
        out_type=jax.ShapeDtypeStruct((batch, D_MODEL), jnp.float32),
        mesh=mesh,
        scratch_types=[
            pltpu.VMEM((b_per_w,), jnp.int32),
            pltpu.VMEM((NBUF, CHUNK, D_MODEL), jnp.float32),
            [pltpu.SemaphoreType.DMA] * NBUF,
            [pltpu.SemaphoreType.DMA] * NBUF,
        ],
    )
    def emb_kernel(table_hbm, idx_hbm, out_hbm, idx_v, rows_v, gsems, wsems):
        wid = lax.axis_index("s") * NC + lax.axis_index("c")
        base = wid * b_per_w
        pltpu.sync_copy(idx_hbm.at[pl.ds(base, b_per_w)], idx_v)

        def fire_gather(c, s):
            pltpu.async_copy(
                table_hbm.at[idx_v.at[pl.ds(c * CHUNK, CHUNK)]],
                rows_v.at[s], gsems[s])

        def drain_gather(s):
            pltpu.make_async_copy(
                table_hbm.at[idx_v.at[pl.ds(0, CHUNK)]],
                rows_v.at[s], gsems[s]).wait()

        def fire_write(c, s):
            pltpu.async_copy(
                rows_v.at[s], out_hbm.at[pl.ds(base + c * CHUNK, CHUNK)],
                wsems[s])

        def drain_write(s):
            pltpu.make_async_copy(
                rows_v.at[s], out_hbm.at[pl.ds(base, CHUNK)],
                wsems[s]).wait()

        def step(c, s, drain_w, fire_next):
            drain_gather(s)
            fire_write(c, s)
            if drain_w:
                drain_write((s + GDEPTH) % NBUF)
            if fire_next:
                fire_gather(c + GDEPTH, (s + GDEPTH) % NBUF)

        for c in range(GDEPTH):
            fire_gather(c, c % NBUF)
        for c in range(NBUF - GDEPTH):
            step(c, c % NBUF, False, True)

        c_main = NBUF - GDEPTH
        n_blocks = (n_chunks - GDEPTH - c_main) // NBUF

        def body(i, carry):
            c0 = c_main + NBUF * i
            for k in range(NBUF):
                step(c0 + k, (c_main + k) % NBUF, True, True)
            return carry

        lax.fori_loop(0, n_blocks, body, 0)
        for c in range(c_main + NBUF * n_blocks, n_chunks):
            step(c, c % NBUF, True, c + GDEPTH < n_chunks)
        for c in range(n_chunks + GDEPTH - NBUF, n_chunks):
            drain_write(c % NBUF)

    return emb_kernel(table, idx_flat)


def kernel(token_ids, weight):
    b, t = token_ids.shape
    idx_flat = token_ids.reshape(b * t).astype(jnp.int32)
    out = _emb_lookup(weight, idx_flat, batch=b * t)
    return out.reshape(b, t, D_MODEL)


# speedup vs baseline: 1.0072x; 1.0072x over previous
import functools

import jax
import jax.numpy as jnp
from jax import lax
from jax.experimental import pallas as pl
from jax.experimental.pallas import tpu as pltpu
from jax.experimental.pallas import tpu_sc as plsc

D_MODEL = 128
NC = 2
NS = 16
NW = NC * NS
CHUNK = 128
NBUF = 7
GDEPTH = 6


@functools.partial(jax.jit, static_argnames=("batch",))
def _emb_lookup(table, idx_flat, *, batch):
    b_per_w = batch // NW
    n_chunks = b_per_w // CHUNK
    assert n_chunks * CHUNK == b_per_w and n_chunks >= 2 * NBUF
    mesh = plsc.VectorSubcoreMesh(
        core_axis_name="c", subcore_axis_name="s",
        num_cores=NC, num_subcores=NS)

    @functools.partial(
        pl.kernel,
        out_type=jax.ShapeDtypeStruct((batch, D_MODEL), jnp.float32),
        mesh=mesh,
        scratch_types=[
            pltpu.VMEM((b_per_w,), jnp.int32),
            pltpu.VMEM((NBUF, CHUNK, D_MODEL), jnp.float32),
            [pltpu.SemaphoreType.DMA] * NBUF,
            [pltpu.SemaphoreType.DMA] * NBUF,
        ],
    )
    def emb_kernel(table_hbm, idx_hbm, out_hbm, idx_v, rows_v, gsems, wsems):
        wid = lax.axis_index("s") * NC + lax.axis_index("c")
        base = wid * b_per_w
        pltpu.sync_copy(idx_hbm.at[pl.ds(base, b_per_w)], idx_v)

        def fire_gather(c, s):
            pltpu.async_copy(
                table_hbm.at[idx_v.at[pl.ds(c * CHUNK, CHUNK)]],
                rows_v.at[s], gsems[s])

        def drain_gather(s):
            pltpu.make_async_copy(
                table_hbm.at[idx_v.at[pl.ds(0, CHUNK)]],
                rows_v.at[s], gsems[s]).wait()

        def fire_write(c, s):
            pltpu.async_copy(
                rows_v.at[s], out_hbm.at[pl.ds(base + c * CHUNK, CHUNK)],
                wsems[s])

        def drain_write(s):
            pltpu.make_async_copy(
                rows_v.at[s], out_hbm.at[pl.ds(base, CHUNK)],
                wsems[s]).wait()

        def step(c, s, drain_w, fire_next):
            drain_gather(s)
            fire_write(c, s)
            if drain_w:
                drain_write((s + GDEPTH) % NBUF)
            if fire_next:
                fire_gather(c + GDEPTH, (s + GDEPTH) % NBUF)

        for c in range(GDEPTH):
            fire_gather(c, c % NBUF)
        for c in range(NBUF - GDEPTH):
            step(c, c % NBUF, False, True)

        c_main = NBUF - GDEPTH
        n_blocks = (n_chunks - GDEPTH - c_main) // NBUF

        def body(i, carry):
            c0 = c_main + NBUF * i
            for k in range(NBUF):
                step(c0 + k, (c_main + k) % NBUF, True, True)
            return carry

        lax.fori_loop(0, n_blocks, body, 0)
        for c in range(c_main + NBUF * n_blocks, n_chunks):
            step(c, c % NBUF, True, c + GDEPTH < n_chunks)
        for c in range(n_chunks + GDEPTH - NBUF, n_chunks):
            drain_write(c % NBUF)

    return emb_kernel(table, idx_flat)


def kernel(token_ids, weight):
    b, t = token_ids.shape
    idx_flat = token_ids.reshape(b * t).astype(jnp.int32)
    out = _emb_lookup(weight, idx_flat, batch=b * t)
    return out.reshape(b, t, D_MODEL)
